# baseline (speedup 1.0000x reference)
import jax
import jax.numpy as jnp
from jax.experimental import pallas as pl
from jax.experimental.pallas import tpu as pltpu

_NUM_SEEDS = 64
_HIDDEN = 2048
_CHUNK = _HIDDEN // _NUM_SEEDS
_NUM_BP = 10
_BT = 1024
_NB = 16384 // _BT
_RING = 3


def _combine_kernel(lc_ref, bp_ref, st_ref, e_ref, bw_ref, x_hbm, o_hbm,
                    xb, ob, rsem, wsem):
    lc = lc_ref[...]
    bp = bp_ref[...]
    st = st_ref[...]

    active = (lc >= 2) & (lc <= 5) & (bp < _NUM_BP)
    act0 = active & (st == 0)
    act1 = active & (st == 1)
    actm = active & (st != 0) & (st != 1)

    one = jnp.float32(1.0)
    half = jnp.float32(0.5)
    m1 = jnp.where(act0, 0.0, jnp.where(act1, one, jnp.where(actm, half, one)))
    m2 = jnp.where(act1, one, jnp.where(actm, half, 0.0))
    m3 = jnp.where(act0, one, 0.0)

    bpc = jnp.clip(bp, 0, _NUM_BP - 1)
    rows = [m1.astype(jnp.float32), m2.astype(jnp.float32), m3.astype(jnp.float32)]
    for r in range(_NUM_BP):
        rows.append((bpc == r).astype(jnp.float32))
    p = jnp.concatenate(rows, axis=0)

    q = jnp.dot(p, e_ref[...], preferred_element_type=jnp.float32)
    w = jnp.sum(q[3:3 + _NUM_BP, :] * bw_ref[...], axis=0, keepdims=True)
    a = q[0:1, :] + w * q[2:3, :]
    b = w * q[1:2, :]

    def read(blk, slot):
        return pltpu.make_async_copy(
            x_hbm.at[pl.ds(blk * _BT, _BT)], xb.at[slot], rsem.at[slot])

    def write(blk, slot):
        return pltpu.make_async_copy(
            ob.at[slot], o_hbm.at[pl.ds(blk * _BT, _BT)], wsem.at[slot])

    for s in range(_RING):
        read(s, s).start()

    for blk in range(_NB):
        slot = blk % _RING
        read(blk, slot).wait()
        if blk >= _RING:
            write(blk - _RING, slot).wait()
        ob[slot] = xb[slot] * a + b
        write(blk, slot).start()
        nxt = blk + _RING
        if nxt < _NB:
            read(nxt, slot).start()

    for blk in range(_NB - _RING, _NB):
        write(blk, blk % _RING).wait()


@jax.jit
def kernel(x, lifecycle_states, blueprint_ids, grafting_strategies, blueprint_weights):
    lc = lifecycle_states.reshape(1, _NUM_SEEDS)
    bp = blueprint_ids.reshape(1, _NUM_SEEDS)
    st = grafting_strategies.reshape(1, _NUM_SEEDS)
    e = (jnp.arange(_HIDDEN, dtype=jnp.int32)[None, :] // _CHUNK
         == jnp.arange(_NUM_SEEDS, dtype=jnp.int32)[:, None]).astype(jnp.float32)

    vspec = pl.BlockSpec(memory_space=pl.ANY)
    return pl.pallas_call(
        _combine_kernel,
        in_specs=[
            pl.BlockSpec((1, _NUM_SEEDS), lambda: (0, 0)),
            pl.BlockSpec((1, _NUM_SEEDS), lambda: (0, 0)),
            pl.BlockSpec((1, _NUM_SEEDS), lambda: (0, 0)),
            pl.BlockSpec((_NUM_SEEDS, _HIDDEN), lambda: (0, 0)),
            pl.BlockSpec((_NUM_BP, _HIDDEN), lambda: (0, 0)),
            vspec,
        ],
        out_specs=vspec,
        out_shape=jax.ShapeDtypeStruct(x.shape, x.dtype),
        scratch_shapes=[
            pltpu.VMEM((_RING, _BT, _HIDDEN), jnp.float32),
            pltpu.VMEM((_RING, _BT, _HIDDEN), jnp.float32),
            pltpu.SemaphoreType.DMA((_RING,)),
            pltpu.SemaphoreType.DMA((_RING,)),
        ],
    )(lc, bp, st, e, blueprint_weights, x)
